# Initial kernel scaffold; baseline (speedup 1.0000x reference)
import jax
import jax.numpy as jnp
from jax.experimental import pallas as pl


def kernel(x, edge_index, W1, b1, Wmu, bmu, Wlv, blv, Wh, bh):
    raise NotImplementedError("write your pallas kernel here")


# speedup vs baseline: 1.3354x; 1.3354x over previous
import functools

import jax
import jax.numpy as jnp
from jax.experimental import pallas as pl

N = 10000
E = 320000
ROW_BLK = 1000


def _matmul_body(x_ref, w_ref, b_ref, r_ref, o_ref, *, act):
    acc = jnp.dot(x_ref[...], w_ref[...], preferred_element_type=jnp.float32)
    acc = acc + r_ref[...] * b_ref[...]
    if act == "relu":
        acc = jnp.maximum(acc, 0.0)
    o_ref[...] = acc


def _matmul(x, w, b, r, act="none"):
    n, k = x.shape
    m = w.shape[1]
    grid = (n // ROW_BLK,)
    return pl.pallas_call(
        functools.partial(_matmul_body, act=act),
        grid=grid,
        in_specs=[
            pl.BlockSpec((ROW_BLK, k), lambda i: (i, 0)),
            pl.BlockSpec((k, m), lambda i: (0, 0)),
            pl.BlockSpec((1, m), lambda i: (0, 0)),
            pl.BlockSpec((ROW_BLK, 1), lambda i: (i, 0)),
        ],
        out_specs=pl.BlockSpec((ROW_BLK, m), lambda i: (i, 0)),
        out_shape=jax.ShapeDtypeStruct((n, m), jnp.float32),
    )(x, w, b.reshape(1, m), r.reshape(n, 1))


DEC_BLK = 200


def _decode_body(zr_ref, zc_ref, wh_ref, bh_ref, adj_ref, task_ref):
    zr = zr_ref[...]
    adj_ref[...] = jax.nn.sigmoid(
        jax.lax.dot_general(zr, zc_ref[...], (((1,), (1,)), ((), ())),
                            preferred_element_type=jnp.float32))
    task_ref[...] = jnp.dot(zr, wh_ref[...],
                            preferred_element_type=jnp.float32) + bh_ref[...]


def _decode(z, wh, bh):
    lat = z.shape[1]
    ncls = wh.shape[1]
    return pl.pallas_call(
        _decode_body,
        grid=(N // DEC_BLK,),
        in_specs=[
            pl.BlockSpec((DEC_BLK, lat), lambda i: (i, 0)),
            pl.BlockSpec((N, lat), lambda i: (0, 0)),
            pl.BlockSpec((lat, ncls), lambda i: (0, 0)),
            pl.BlockSpec((1, ncls), lambda i: (0, 0)),
        ],
        out_specs=[
            pl.BlockSpec((DEC_BLK, N), lambda i: (i, 0)),
            pl.BlockSpec((DEC_BLK, ncls), lambda i: (i, 0)),
        ],
        out_shape=[
            jax.ShapeDtypeStruct((N, N), jnp.float32),
            jax.ShapeDtypeStruct((N, ncls), jnp.float32),
        ],
    )(z, z, wh, bh.reshape(1, ncls))


def _csrf_flow(src, dst, num_iters):
    w = jnp.ones((E,), jnp.float32)
    snap = []
    for _ in range(num_iters):
        deg = (jax.ops.segment_sum(w, src, num_segments=N)
               + jax.ops.segment_sum(w, dst, num_segments=N))
        kappa = 4.0 - deg[src] - deg[dst]
        w = w * (1.0 - 0.05 * jnp.tanh(0.1 * kappa))
        w = w * (E / jnp.maximum(w.sum(), 1e-12))
        w = jnp.clip(w, 1e-6, None)
        snap.append(w)
    return snap


def _graph_norm(src, dst, ew):
    deg = jax.ops.segment_sum(ew, dst, num_segments=N) + 1.0
    dinv = jax.lax.rsqrt(deg)
    norm = dinv[src] * ew * dinv[dst]
    r = jax.ops.segment_sum(norm, dst, num_segments=N) + dinv * dinv
    return dinv, norm, r


def _propagate(x, src, dst, norm, dinv):
    edge = jax.ops.segment_sum(x[src] * norm[:, None], dst, num_segments=N)
    return edge + (dinv * dinv)[:, None] * x


def kernel(x, edge_index, W1, b1, Wmu, bmu, Wlv, blv, Wh, bh):
    src = edge_index[0]
    dst = edge_index[1]

    snaps = _csrf_flow(src, dst, 5)
    ew1, ew5 = snaps[0], snaps[4]

    Wc = jnp.concatenate([Wmu, Wlv], axis=1)

    dinv, norm, r = _graph_norm(src, dst, jnp.ones((E,), jnp.float32))
    aggx = _propagate(x, src, dst, norm, dinv)
    h = _matmul(aggx, W1, b1, r, act="relu")
    hc = _matmul(h, Wc, jnp.zeros((Wc.shape[1],), jnp.float32), r)
    agg_hc = _propagate(hc, src, dst, norm, dinv)
    mu = agg_hc[:, : Wmu.shape[1]] + r[:, None] * bmu
    logvar = agg_hc[:, Wmu.shape[1]:] + r[:, None] * blv
    z = mu

    def encode_mu(ew):
        dinv_g, norm_g, r_g = _graph_norm(src, dst, ew)
        aggx_g = _propagate(x, src, dst, norm_g, dinv_g)
        h_g = _matmul(aggx_g, W1, b1, r_g, act="relu")
        hm_g = _matmul(h_g, Wmu, jnp.zeros((Wmu.shape[1],), jnp.float32), r_g)
        return _propagate(hm_g, src, dst, norm_g, dinv_g) + r_g[:, None] * bmu

    z1 = encode_mu(ew1)
    z5 = encode_mu(ew5)

    adj_pred, task_pred = _decode(z, Wh, bh)
    return (z, mu, logvar, z1, z5, adj_pred, task_pred)


# speedup vs baseline: 1.4675x; 1.0990x over previous
import functools

import jax
import jax.numpy as jnp
from jax import lax
from jax.experimental import pallas as pl
from jax.experimental.pallas import tpu as pltpu
from jax.experimental.pallas import tpu_sc as plsc

N = 10000
E = 320000
ROW_BLK = 1000

NC, NS, LANES = 2, 16, 16
NW = NC * NS
CHUNK = 128
CPW = 80
EPW = CHUNK * CPW
E_PAD = EPW * NW
NPAD = 10112


def _prop_body(x_hbm, srcb_hbm, dstb_hbm, wb_hbm, zeros_hbm, out_hbm,
               src_v, dst_v, w_v, rows_v, gsem, acc_sh, *, ch):
    c = lax.axis_index("c")
    s = lax.axis_index("s")
    wid = s * NC + c
    rpt = NPAD // NS

    pltpu.sync_copy(zeros_hbm.at[pl.ds(s * rpt, rpt)],
                    acc_sh.at[pl.ds(s * rpt, rpt)])
    pltpu.sync_copy(srcb_hbm.at[pl.ds(wid * CPW, CPW)], src_v)
    pltpu.sync_copy(dstb_hbm.at[pl.ds(wid * CPW, CPW)], dst_v)
    pltpu.sync_copy(wb_hbm.at[pl.ds(wid * CPW, CPW)], w_v)
    plsc.subcore_barrier()

    def chunk(j, carry):
        pltpu.async_copy(x_hbm.at[src_v.at[j]], rows_v, gsem).wait()
        for i0 in range(0, CHUNK, LANES):
            w16 = w_v[j, pl.ds(i0, LANES)]
            for k in range(LANES):
                w = w16[k]
                for g in range(ch // LANES):
                    sl = pl.ds(g * LANES, LANES)
                    rows_v[i0 + k, sl] = rows_v[i0 + k, sl] * w
        pltpu.sync_copy(rows_v, acc_sh.at[dst_v.at[j]], add=True)
        return carry

    lax.fori_loop(0, CPW, chunk, 0)
    plsc.subcore_barrier()
    pltpu.sync_copy(acc_sh.at[pl.ds(s * rpt, rpt)],
                    out_hbm.at[pl.ds(c * NPAD + s * rpt, rpt)])


@functools.partial(jax.jit, static_argnames=("ch",))
def _propagate_sc(x, srcb, dstb, wb, *, ch):
    zeros = jnp.zeros((NPAD, ch), jnp.float32)
    mesh = plsc.VectorSubcoreMesh(core_axis_name="c", subcore_axis_name="s")
    fn = pl.kernel(
        functools.partial(_prop_body, ch=ch),
        out_type=jax.ShapeDtypeStruct((NC * NPAD, ch), jnp.float32),
        mesh=mesh,
        scratch_types=[
            pltpu.VMEM((CPW, CHUNK), jnp.int32),
            pltpu.VMEM((CPW, CHUNK), jnp.int32),
            pltpu.VMEM((CPW, CHUNK), jnp.float32),
            pltpu.VMEM((CHUNK, ch), jnp.float32),
            pltpu.SemaphoreType.DMA,
            pltpu.VMEM_SHARED((NPAD, ch), jnp.float32),
        ],
        compiler_params=pltpu.CompilerParams(use_tc_tiling_on_sc=False),
    )
    parts = fn(x, srcb, dstb, wb, zeros)
    return parts[:N] + parts[NPAD:NPAD + N]


def _pad_edges(src, dst):
    pad = E_PAD - E
    srcp = jnp.concatenate([src, jnp.zeros((pad,), jnp.int32)])
    dstp = jnp.concatenate([dst, jnp.full((pad,), N, jnp.int32)])
    return srcp.reshape(E_PAD // CHUNK, CHUNK), dstp.reshape(E_PAD // CHUNK, CHUNK)


def _pad_w(w):
    return jnp.concatenate(
        [w, jnp.zeros((E_PAD - E,), jnp.float32)]).reshape(E_PAD // CHUNK, CHUNK)


def _matmul_body(x_ref, w_ref, b_ref, r_ref, o_ref, *, act):
    acc = jnp.dot(x_ref[...], w_ref[...], preferred_element_type=jnp.float32)
    acc = acc + r_ref[...] * b_ref[...]
    if act == "relu":
        acc = jnp.maximum(acc, 0.0)
    o_ref[...] = acc


def _matmul(x, w, b, r, act="none"):
    n, k = x.shape
    m = w.shape[1]
    grid = (n // ROW_BLK,)
    return pl.pallas_call(
        functools.partial(_matmul_body, act=act),
        grid=grid,
        in_specs=[
            pl.BlockSpec((ROW_BLK, k), lambda i: (i, 0)),
            pl.BlockSpec((k, m), lambda i: (0, 0)),
            pl.BlockSpec((1, m), lambda i: (0, 0)),
            pl.BlockSpec((ROW_BLK, 1), lambda i: (i, 0)),
        ],
        out_specs=pl.BlockSpec((ROW_BLK, m), lambda i: (i, 0)),
        out_shape=jax.ShapeDtypeStruct((n, m), jnp.float32),
    )(x, w, b.reshape(1, m), r.reshape(n, 1))


DEC_BLK = 200


def _decode_body(zr_ref, zc_ref, wh_ref, bh_ref, adj_ref, task_ref):
    zr = zr_ref[...]
    adj_ref[...] = jax.nn.sigmoid(
        jax.lax.dot_general(zr, zc_ref[...], (((1,), (1,)), ((), ())),
                            preferred_element_type=jnp.float32))
    task_ref[...] = jnp.dot(zr, wh_ref[...],
                            preferred_element_type=jnp.float32) + bh_ref[...]


def _decode(z, wh, bh):
    lat = z.shape[1]
    ncls = wh.shape[1]
    return pl.pallas_call(
        _decode_body,
        grid=(N // DEC_BLK,),
        in_specs=[
            pl.BlockSpec((DEC_BLK, lat), lambda i: (i, 0)),
            pl.BlockSpec((N, lat), lambda i: (0, 0)),
            pl.BlockSpec((lat, ncls), lambda i: (0, 0)),
            pl.BlockSpec((1, ncls), lambda i: (0, 0)),
        ],
        out_specs=[
            pl.BlockSpec((DEC_BLK, N), lambda i: (i, 0)),
            pl.BlockSpec((DEC_BLK, ncls), lambda i: (i, 0)),
        ],
        out_shape=[
            jax.ShapeDtypeStruct((N, N), jnp.float32),
            jax.ShapeDtypeStruct((N, ncls), jnp.float32),
        ],
    )(z, z, wh, bh.reshape(1, ncls))


def _csrf_flow(src, dst, num_iters):
    w = jnp.ones((E,), jnp.float32)
    snap = []
    for _ in range(num_iters):
        deg = (jax.ops.segment_sum(w, src, num_segments=N)
               + jax.ops.segment_sum(w, dst, num_segments=N))
        kappa = 4.0 - deg[src] - deg[dst]
        w = w * (1.0 - 0.05 * jnp.tanh(0.1 * kappa))
        w = w * (E / jnp.maximum(w.sum(), 1e-12))
        w = jnp.clip(w, 1e-6, None)
        snap.append(w)
    return snap


def _graph_norm(src, dst, ew):
    deg = jax.ops.segment_sum(ew, dst, num_segments=N) + 1.0
    dinv = jax.lax.rsqrt(deg)
    norm = dinv[src] * ew * dinv[dst]
    r = jax.ops.segment_sum(norm, dst, num_segments=N) + dinv * dinv
    return dinv, norm, r


def _propagate(x, srcb, dstb, normb, dinv):
    edge = _propagate_sc(x, srcb, dstb, normb, ch=x.shape[1])
    return edge + (dinv * dinv)[:, None] * x


def kernel(x, edge_index, W1, b1, Wmu, bmu, Wlv, blv, Wh, bh):
    src = edge_index[0]
    dst = edge_index[1]
    srcb, dstb = _pad_edges(src, dst)

    snaps = _csrf_flow(src, dst, 5)
    ew1, ew5 = snaps[0], snaps[4]

    Wc = jnp.concatenate([Wmu, Wlv], axis=1)

    dinv, norm, r = _graph_norm(src, dst, jnp.ones((E,), jnp.float32))
    normb = _pad_w(norm)
    aggx = _propagate(x, srcb, dstb, normb, dinv)
    h = _matmul(aggx, W1, b1, r, act="relu")
    hc = _matmul(h, Wc, jnp.zeros((Wc.shape[1],), jnp.float32), r)
    agg_hc = _propagate(hc, srcb, dstb, normb, dinv)
    mu = agg_hc[:, : Wmu.shape[1]] + r[:, None] * bmu
    logvar = agg_hc[:, Wmu.shape[1]:] + r[:, None] * blv
    z = mu

    def encode_mu(ew):
        dinv_g, norm_g, r_g = _graph_norm(src, dst, ew)
        normb_g = _pad_w(norm_g)
        aggx_g = _propagate(x, srcb, dstb, normb_g, dinv_g)
        h_g = _matmul(aggx_g, W1, b1, r_g, act="relu")
        hm_g = _matmul(h_g, Wmu, jnp.zeros((Wmu.shape[1],), jnp.float32), r_g)
        return _propagate(hm_g, srcb, dstb, normb_g, dinv_g) + r_g[:, None] * bmu

    z1 = encode_mu(ew1)
    z5 = encode_mu(ew5)

    adj_pred, task_pred = _decode(z, Wh, bh)
    return (z, mu, logvar, z1, z5, adj_pred, task_pred)


# speedup vs baseline: 14.6579x; 9.9882x over previous
import functools

import jax
import jax.numpy as jnp
from jax import lax
from jax.experimental import pallas as pl
from jax.experimental.pallas import tpu as pltpu
from jax.experimental.pallas import tpu_sc as plsc

N = 10000
E = 320000
ROW_BLK = 1000

NC, NS, LANES = 2, 16, 16
NW = NC * NS
CHUNK = 128
CPW = 80
EPW = CHUNK * CPW
E_PAD = EPW * NW
NPAD = 10240
NROWS = E_PAD // CHUNK
NREAL = E // CHUNK
CPT = NROWS // NS
SLICE = NPAD // NS


def _prop_body(x_hbm, srcb_hbm, dstb_hbm, wb_hbm, zeros_hbm, out_hbm,
               src_v, dst_v, w_v, rows_v, gsem, acc_sh, *, ch):
    c = lax.axis_index("c")
    s = lax.axis_index("s")
    wid = s * NC + c
    rpt = NPAD // NS

    pltpu.sync_copy(zeros_hbm.at[pl.ds(s * rpt, rpt)],
                    acc_sh.at[pl.ds(s * rpt, rpt)])
    pltpu.sync_copy(srcb_hbm.at[pl.ds(wid * CPW, CPW)], src_v)
    pltpu.sync_copy(dstb_hbm.at[pl.ds(wid * CPW, CPW)], dst_v)
    pltpu.sync_copy(wb_hbm.at[pl.ds(wid * CPW, CPW)], w_v)
    plsc.subcore_barrier()

    def chunk(j, carry):
        pltpu.async_copy(x_hbm.at[src_v.at[j]], rows_v, gsem).wait()
        for i0 in range(0, CHUNK, LANES):
            w16 = w_v[j, pl.ds(i0, LANES)]
            for k in range(LANES):
                w = w16[k]
                for g in range(ch // LANES):
                    sl = pl.ds(g * LANES, LANES)
                    rows_v[i0 + k, sl] = rows_v[i0 + k, sl] * w
        pltpu.sync_copy(rows_v, acc_sh.at[dst_v.at[j]], add=True)
        return carry

    lax.fori_loop(0, CPW, chunk, 0)
    plsc.subcore_barrier()
    pltpu.sync_copy(acc_sh.at[pl.ds(s * rpt, rpt)],
                    out_hbm.at[pl.ds(c * NPAD + s * rpt, rpt)])


@functools.partial(jax.jit, static_argnames=("ch",))
def _propagate_sc(x, srcb, dstb, wb, *, ch):
    zeros = jnp.zeros((NPAD, ch), jnp.float32)
    mesh = plsc.VectorSubcoreMesh(core_axis_name="c", subcore_axis_name="s")
    fn = pl.kernel(
        functools.partial(_prop_body, ch=ch),
        out_type=jax.ShapeDtypeStruct((NC * NPAD, ch), jnp.float32),
        mesh=mesh,
        scratch_types=[
            pltpu.VMEM((CPW, CHUNK), jnp.int32),
            pltpu.VMEM((CPW, CHUNK), jnp.int32),
            pltpu.VMEM((CPW, CHUNK), jnp.float32),
            pltpu.VMEM((CHUNK, ch), jnp.float32),
            pltpu.SemaphoreType.DMA,
            pltpu.VMEM_SHARED((NPAD, ch), jnp.float32),
        ],
        compiler_params=pltpu.CompilerParams(use_tc_tiling_on_sc=False),
    )
    parts = fn(x, srcb, dstb, wb, zeros)
    return parts[:N] + parts[NPAD:NPAD + N]


def _pad_edges(src, dst):
    pad = E_PAD - E
    srcp = jnp.concatenate([src, jnp.zeros((pad,), jnp.int32)])
    dstp = jnp.concatenate([dst, jnp.full((pad,), N, jnp.int32)])
    return srcp.reshape(E_PAD // CHUNK, CHUNK), dstp.reshape(E_PAD // CHUNK, CHUNK)


def _matmul_body(x_ref, w_ref, b_ref, r_ref, o_ref, *, act):
    acc = jnp.dot(x_ref[...], w_ref[...], preferred_element_type=jnp.float32)
    acc = acc + r_ref[...] * b_ref[...]
    if act == "relu":
        acc = jnp.maximum(acc, 0.0)
    o_ref[...] = acc


def _matmul(x, w, b, r, act="none"):
    n, k = x.shape
    m = w.shape[1]
    grid = (n // ROW_BLK,)
    return pl.pallas_call(
        functools.partial(_matmul_body, act=act),
        grid=grid,
        in_specs=[
            pl.BlockSpec((ROW_BLK, k), lambda i: (i, 0)),
            pl.BlockSpec((k, m), lambda i: (0, 0)),
            pl.BlockSpec((1, m), lambda i: (0, 0)),
            pl.BlockSpec((ROW_BLK, 1), lambda i: (i, 0)),
        ],
        out_specs=pl.BlockSpec((ROW_BLK, m), lambda i: (i, 0)),
        out_shape=jax.ShapeDtypeStruct((n, m), jnp.float32),
    )(x, w, b.reshape(1, m), r.reshape(n, 1))


DEC_BLK = 200


def _decode_body(zr_ref, zc_ref, wh_ref, bh_ref, adj_ref, task_ref):
    zr = zr_ref[...]
    adj_ref[...] = jax.nn.sigmoid(
        jax.lax.dot_general(zr, zc_ref[...], (((1,), (1,)), ((), ())),
                            preferred_element_type=jnp.float32))
    task_ref[...] = jnp.dot(zr, wh_ref[...],
                            preferred_element_type=jnp.float32) + bh_ref[...]


def _decode(z, wh, bh):
    lat = z.shape[1]
    ncls = wh.shape[1]
    return pl.pallas_call(
        _decode_body,
        grid=(N // DEC_BLK,),
        in_specs=[
            pl.BlockSpec((DEC_BLK, lat), lambda i: (i, 0)),
            pl.BlockSpec((N, lat), lambda i: (0, 0)),
            pl.BlockSpec((lat, ncls), lambda i: (0, 0)),
            pl.BlockSpec((1, ncls), lambda i: (0, 0)),
        ],
        out_specs=[
            pl.BlockSpec((DEC_BLK, N), lambda i: (i, 0)),
            pl.BlockSpec((DEC_BLK, ncls), lambda i: (i, 0)),
        ],
        out_shape=[
            jax.ShapeDtypeStruct((N, N), jnp.float32),
            jax.ShapeDtypeStruct((N, ncls), jnp.float32),
        ],
    )(z, z, wh, bh.reshape(1, ncls))


def _rsqrt16(x):
    i = plsc.bitcast(x, jnp.int32)
    i = jnp.int32(0x5F3759DF) - lax.shift_right_logical(i, 1)
    y = plsc.bitcast(i, jnp.float32)
    for _ in range(3):
        y = y * (1.5 - 0.5 * x * y * y)
    return y


def _scalar_body(srcb, dstb, dinv3, r3, norm3,
                 src_t, dst_t, w_t, w1_t, acc, full, red, rbuf, nbuf, ssum,
                 part_sh, red_sh, sums_sh):
    s = lax.axis_index("s")
    base_row = s * CPT

    pltpu.sync_copy(srcb.at[pl.ds(base_row, CPT)], src_t)
    pltpu.sync_copy(dstb.at[pl.ds(base_row, CPT)], dst_t)

    def zero_acc():
        def zb(i, _):
            acc[pl.ds(i * LANES, LANES)] = jnp.zeros((LANES,), jnp.float32)
            return 0
        lax.fori_loop(0, NPAD // LANES, zb, 0)

    def real_of(j):
        return jnp.where(base_row + j < NREAL, 1.0, 0.0).astype(jnp.float32)

    def stage_reduce(group_fn):
        pltpu.sync_copy(acc, part_sh.at[s])
        plsc.subcore_barrier()
        for t in range(NS):
            pltpu.sync_copy(part_sh.at[t, pl.ds(s * SLICE, SLICE)], red.at[t])

        def rb(g, _):
            v = red[0, pl.ds(g * LANES, LANES)]
            for t in range(1, NS):
                v = v + red[t, pl.ds(g * LANES, LANES)]
            rbuf[pl.ds(g * LANES, LANES)] = group_fn(g, v)
            return 0
        lax.fori_loop(0, SLICE // LANES, rb, 0)
        plsc.subcore_barrier()

    def broadcast_rbuf():
        pltpu.sync_copy(rbuf, red_sh.at[pl.ds(s * SLICE, SLICE)])
        plsc.subcore_barrier()
        pltpu.sync_copy(red_sh, full)
        plsc.subcore_barrier()

    def init_w(j, _):
        r = real_of(j)
        for g in range(CHUNK // LANES):
            sl = pl.ds(g * LANES, LANES)
            w_t[j, sl] = jnp.full((LANES,), 1.0) * r
        return 0
    lax.fori_loop(0, CPT, init_w, 0)

    for it in range(5):
        zero_acc()

        def sc_body(j, _):
            for g in range(CHUNK // LANES):
                sl = pl.ds(g * LANES, LANES)
                w16 = w_t[j, sl]
                plsc.addupdate_scatter(acc, [src_t[j, sl]], w16)
                plsc.addupdate_scatter(acc, [dst_t[j, sl]], w16)
            return 0
        lax.fori_loop(0, CPT, sc_body, 0)
        stage_reduce(lambda g, v: v)
        broadcast_rbuf()

        def k_body(j, vs):
            for g in range(CHUNK // LANES):
                sl = pl.ds(g * LANES, LANES)
                a = plsc.load_gather(full, [src_t[j, sl]])
                b = plsc.load_gather(full, [dst_t[j, sl]])
                e2 = jnp.exp(0.2 * (4.0 - a - b))
                th = (e2 - 1.0) / (e2 + 1.0)
                w16 = w_t[j, sl] * (1.0 - 0.05 * th)
                w_t[j, sl] = w16
                vs = vs + w16
            return vs
        vsum = lax.fori_loop(0, CPT, k_body, jnp.zeros((LANES,), jnp.float32))

        ssum[0, :] = vsum
        pltpu.sync_copy(ssum.at[0], sums_sh.at[s])
        plsc.subcore_barrier()
        pltpu.sync_copy(sums_sh, ssum)
        tot16 = ssum[0, :]
        for t in range(1, NS):
            tot16 = tot16 + ssum[t, :]
        total = jnp.sum(tot16)
        scale = (jnp.full((LANES,), float(E), jnp.float32)
                 / jnp.maximum(jnp.broadcast_to(total, (LANES,)), 1e-12))
        plsc.subcore_barrier()

        def renorm(j, _):
            r = real_of(j)
            for g in range(CHUNK // LANES):
                sl = pl.ds(g * LANES, LANES)
                w16 = jnp.maximum(w_t[j, sl] * scale, 1e-6) * r
                w_t[j, sl] = w16
                if it == 0:
                    w1_t[j, sl] = w16
            return 0
        lax.fori_loop(0, CPT, renorm, 0)

    for gi in range(3):
        def ew16(j, sl):
            if gi == 0:
                return jnp.full((LANES,), 1.0) * real_of(j)
            return (w1_t if gi == 1 else w_t)[j, sl]

        zero_acc()

        def d_body(j, _):
            for g in range(CHUNK // LANES):
                sl = pl.ds(g * LANES, LANES)
                plsc.addupdate_scatter(acc, [dst_t[j, sl]], ew16(j, sl))
            return 0
        lax.fori_loop(0, CPT, d_body, 0)
        stage_reduce(lambda g, v: _rsqrt16(v + 1.0))
        pltpu.sync_copy(rbuf, dinv3.at[gi].at[pl.ds(s * SLICE, SLICE)])
        broadcast_rbuf()

        zero_acc()

        def n_body(jj, _):
            for k in range(LANES):
                j = jj * LANES + k
                for g in range(CHUNK // LANES):
                    sl = pl.ds(g * LANES, LANES)
                    d16 = dst_t[j, sl]
                    a = plsc.load_gather(full, [src_t[j, sl]])
                    b = plsc.load_gather(full, [d16])
                    nm = a * ew16(j, sl) * b
                    nbuf[k, sl] = nm
                    plsc.addupdate_scatter(acc, [d16], nm)
            pltpu.sync_copy(
                nbuf, norm3.at[gi].at[pl.ds(base_row + jj * LANES, LANES)])
            return 0
        lax.fori_loop(0, CPT // LANES, n_body, 0)

        def r_group(g, v):
            d16 = full[pl.ds(s * SLICE + g * LANES, LANES)]
            return v + d16 * d16
        stage_reduce(r_group)
        pltpu.sync_copy(rbuf, r3.at[gi].at[pl.ds(s * SLICE, SLICE)])
        plsc.subcore_barrier()


@jax.jit
def _scalar_sc(srcb, dstb):
    mesh = plsc.VectorSubcoreMesh(
        core_axis_name="c", subcore_axis_name="s", num_cores=1)
    fn = pl.kernel(
        _scalar_body,
        out_type=(
            jax.ShapeDtypeStruct((3, NPAD), jnp.float32),
            jax.ShapeDtypeStruct((3, NPAD), jnp.float32),
            jax.ShapeDtypeStruct((3, NROWS, CHUNK), jnp.float32),
        ),
        mesh=mesh,
        scratch_types=[
            pltpu.VMEM((CPT, CHUNK), jnp.int32),
            pltpu.VMEM((CPT, CHUNK), jnp.int32),
            pltpu.VMEM((CPT, CHUNK), jnp.float32),
            pltpu.VMEM((CPT, CHUNK), jnp.float32),
            pltpu.VMEM((NPAD,), jnp.float32),
            pltpu.VMEM((NPAD,), jnp.float32),
            pltpu.VMEM((NS, SLICE), jnp.float32),
            pltpu.VMEM((SLICE,), jnp.float32),
            pltpu.VMEM((LANES, CHUNK), jnp.float32),
            pltpu.VMEM((NS, LANES), jnp.float32),
            pltpu.VMEM_SHARED((NS, NPAD), jnp.float32),
            pltpu.VMEM_SHARED((NPAD,), jnp.float32),
            pltpu.VMEM_SHARED((NS, LANES), jnp.float32),
        ],
        compiler_params=pltpu.CompilerParams(
            use_tc_tiling_on_sc=False, needs_layout_passes=False),
    )
    return fn(srcb, dstb)


def _propagate(x, srcb, dstb, normb, dinv):
    edge = _propagate_sc(x, srcb, dstb, normb, ch=x.shape[1])
    return edge + (dinv * dinv)[:, None] * x


def kernel(x, edge_index, W1, b1, Wmu, bmu, Wlv, blv, Wh, bh):
    src = edge_index[0]
    dst = edge_index[1]
    srcb, dstb = _pad_edges(src, dst)

    dinv3, r3, norm3 = _scalar_sc(srcb, dstb)

    Wc = jnp.concatenate([Wmu, Wlv], axis=1)

    dinv, r, normb = dinv3[0, :N], r3[0, :N], norm3[0]
    aggx = _propagate(x, srcb, dstb, normb, dinv)
    h = _matmul(aggx, W1, b1, r, act="relu")
    hc = _matmul(h, Wc, jnp.zeros((Wc.shape[1],), jnp.float32), r)
    agg_hc = _propagate(hc, srcb, dstb, normb, dinv)
    mu = agg_hc[:, : Wmu.shape[1]] + r[:, None] * bmu
    logvar = agg_hc[:, Wmu.shape[1]:] + r[:, None] * blv
    z = mu

    def encode_mu(gi):
        dinv_g, r_g, normb_g = dinv3[gi, :N], r3[gi, :N], norm3[gi]
        aggx_g = _propagate(x, srcb, dstb, normb_g, dinv_g)
        h_g = _matmul(aggx_g, W1, b1, r_g, act="relu")
        hm_g = _matmul(h_g, Wmu, jnp.zeros((Wmu.shape[1],), jnp.float32), r_g)
        return _propagate(hm_g, srcb, dstb, normb_g, dinv_g) + r_g[:, None] * bmu

    z1 = encode_mu(1)
    z5 = encode_mu(2)

    adj_pred, task_pred = _decode(z, Wh, bh)
    return (z, mu, logvar, z1, z5, adj_pred, task_pred)


# speedup vs baseline: 16.6042x; 1.1328x over previous
import functools

import jax
import jax.numpy as jnp
from jax import lax
from jax.experimental import pallas as pl
from jax.experimental.pallas import tpu as pltpu
from jax.experimental.pallas import tpu_sc as plsc

N = 10000
E = 320000
ROW_BLK = 1000

NC, NS, LANES = 2, 16, 16
NW = NC * NS
CHUNK = 128
CPW = 80
EPW = CHUNK * CPW
E_PAD = EPW * NW
NPAD = 10240
NROWS = E_PAD // CHUNK
NREAL = E // CHUNK
CPT = NROWS // NS
SLICE = NPAD // NS


def _prop_body(x_hbm, srcb_hbm, dstb_hbm, wb_hbm, zeros_hbm, out_hbm,
               src_v, dst_v, w_v, rows0, rows1, gs0, gs1, acc_sh,
               *, ch):
    c = lax.axis_index("c")
    s = lax.axis_index("s")
    wid = s * NC + c
    rpt = NPAD // NS

    pltpu.sync_copy(zeros_hbm.at[pl.ds(s * rpt, rpt)],
                    acc_sh.at[pl.ds(s * rpt, rpt)])
    pltpu.sync_copy(srcb_hbm.at[pl.ds(wid * CPW, CPW)], src_v)
    pltpu.sync_copy(dstb_hbm.at[pl.ds(wid * CPW, CPW)], dst_v)
    pltpu.sync_copy(wb_hbm.at[pl.ds(wid * CPW, CPW)], w_v)
    plsc.subcore_barrier()

    def mul(rows, j):
        for i0 in range(0, CHUNK, LANES):
            w16 = w_v[j, pl.ds(i0, LANES)]
            for k in range(LANES):
                w = w16[k]
                for g in range(ch // LANES):
                    sl = pl.ds(g * LANES, LANES)
                    rows[i0 + k, sl] = rows[i0 + k, sl] * w

    def gather(j, rows, sem):
        return pltpu.async_copy(x_hbm.at[src_v.at[j]], rows, sem)

    gather(0, rows0, gs0)
    gather(1, rows1, gs1)

    def pair(jj, carry):
        j0 = jj * 2
        j1 = j0 + 1
        pltpu.make_async_copy(x_hbm.at[src_v.at[j0]], rows0, gs0).wait()
        mul(rows0, j0)
        pltpu.sync_copy(rows0, acc_sh.at[dst_v.at[j0]], add=True)

        @pl.when(j0 + 2 < CPW)
        def _():
            gather(j0 + 2, rows0, gs0)

        pltpu.make_async_copy(x_hbm.at[src_v.at[j1]], rows1, gs1).wait()
        mul(rows1, j1)
        pltpu.sync_copy(rows1, acc_sh.at[dst_v.at[j1]], add=True)

        @pl.when(j1 + 2 < CPW)
        def _():
            gather(j1 + 2, rows1, gs1)
        return carry

    lax.fori_loop(0, CPW // 2, pair, 0)
    plsc.subcore_barrier()
    pltpu.sync_copy(acc_sh.at[pl.ds(s * rpt, rpt)],
                    out_hbm.at[pl.ds(c * NPAD + s * rpt, rpt)])


@functools.partial(jax.jit, static_argnames=("ch",))
def _propagate_sc(x, srcb, dstb, wb, *, ch):
    zeros = jnp.zeros((NPAD, ch), jnp.float32)
    mesh = plsc.VectorSubcoreMesh(core_axis_name="c", subcore_axis_name="s")
    fn = pl.kernel(
        functools.partial(_prop_body, ch=ch),
        out_type=jax.ShapeDtypeStruct((NC * NPAD, ch), jnp.float32),
        mesh=mesh,
        scratch_types=[
            pltpu.VMEM((CPW, CHUNK), jnp.int32),
            pltpu.VMEM((CPW, CHUNK), jnp.int32),
            pltpu.VMEM((CPW, CHUNK), jnp.float32),
            pltpu.VMEM((CHUNK, ch), jnp.float32),
            pltpu.VMEM((CHUNK, ch), jnp.float32),
            pltpu.SemaphoreType.DMA,
            pltpu.SemaphoreType.DMA,
            pltpu.VMEM_SHARED((NPAD, ch), jnp.float32),
        ],
        compiler_params=pltpu.CompilerParams(use_tc_tiling_on_sc=False),
    )
    parts = fn(x, srcb, dstb, wb, zeros)
    return parts[:N] + parts[NPAD:NPAD + N]


def _pad_edges(src, dst):
    pad = E_PAD - E
    srcp = jnp.concatenate([src, jnp.zeros((pad,), jnp.int32)])
    dstp = jnp.concatenate([dst, jnp.full((pad,), N, jnp.int32)])
    return srcp.reshape(E_PAD // CHUNK, CHUNK), dstp.reshape(E_PAD // CHUNK, CHUNK)


def _matmul_body(x_ref, w_ref, b_ref, r_ref, o_ref, *, act):
    acc = jnp.dot(x_ref[...], w_ref[...], preferred_element_type=jnp.float32)
    acc = acc + r_ref[...] * b_ref[...]
    if act == "relu":
        acc = jnp.maximum(acc, 0.0)
    o_ref[...] = acc


def _matmul(x, w, b, r, act="none"):
    n, k = x.shape
    m = w.shape[1]
    grid = (n // ROW_BLK,)
    return pl.pallas_call(
        functools.partial(_matmul_body, act=act),
        grid=grid,
        in_specs=[
            pl.BlockSpec((ROW_BLK, k), lambda i: (i, 0)),
            pl.BlockSpec((k, m), lambda i: (0, 0)),
            pl.BlockSpec((1, m), lambda i: (0, 0)),
            pl.BlockSpec((ROW_BLK, 1), lambda i: (i, 0)),
        ],
        out_specs=pl.BlockSpec((ROW_BLK, m), lambda i: (i, 0)),
        out_shape=jax.ShapeDtypeStruct((n, m), jnp.float32),
    )(x, w, b.reshape(1, m), r.reshape(n, 1))


DEC_BLK = 200


def _decode_body(zr_ref, zc_ref, wh_ref, bh_ref, adj_ref, task_ref):
    zr = zr_ref[...]
    adj_ref[...] = jax.nn.sigmoid(
        jax.lax.dot_general(zr, zc_ref[...], (((1,), (1,)), ((), ())),
                            preferred_element_type=jnp.float32))
    task_ref[...] = jnp.dot(zr, wh_ref[...],
                            preferred_element_type=jnp.float32) + bh_ref[...]


def _decode(z, wh, bh):
    lat = z.shape[1]
    ncls = wh.shape[1]
    return pl.pallas_call(
        _decode_body,
        grid=(N // DEC_BLK,),
        in_specs=[
            pl.BlockSpec((DEC_BLK, lat), lambda i: (i, 0)),
            pl.BlockSpec((N, lat), lambda i: (0, 0)),
            pl.BlockSpec((lat, ncls), lambda i: (0, 0)),
            pl.BlockSpec((1, ncls), lambda i: (0, 0)),
        ],
        out_specs=[
            pl.BlockSpec((DEC_BLK, N), lambda i: (i, 0)),
            pl.BlockSpec((DEC_BLK, ncls), lambda i: (i, 0)),
        ],
        out_shape=[
            jax.ShapeDtypeStruct((N, N), jnp.float32),
            jax.ShapeDtypeStruct((N, ncls), jnp.float32),
        ],
    )(z, z, wh, bh.reshape(1, ncls))


def _rsqrt16(x):
    i = plsc.bitcast(x, jnp.int32)
    i = jnp.int32(0x5F3759DF) - lax.shift_right_logical(i, 1)
    y = plsc.bitcast(i, jnp.float32)
    for _ in range(3):
        y = y * (1.5 - 0.5 * x * y * y)
    return y


def _scalar_body(srcb, dstb, dinv3, r3, norm3,
                 src_t, dst_t, w_t, w1_t, acc, full, red, rbuf, nbuf, ssum,
                 part_sh, red_sh, sums_sh):
    s = lax.axis_index("s")
    base_row = s * CPT

    pltpu.sync_copy(srcb.at[pl.ds(base_row, CPT)], src_t)
    pltpu.sync_copy(dstb.at[pl.ds(base_row, CPT)], dst_t)

    def zero_acc():
        def zb(i, _):
            acc[pl.ds(i * LANES, LANES)] = jnp.zeros((LANES,), jnp.float32)
            return 0
        lax.fori_loop(0, NPAD // LANES, zb, 0)

    def real_of(j):
        return jnp.where(base_row + j < NREAL, 1.0, 0.0).astype(jnp.float32)

    def stage_reduce(group_fn):
        pltpu.sync_copy(acc, part_sh.at[s])
        plsc.subcore_barrier()
        for t in range(NS):
            pltpu.sync_copy(part_sh.at[t, pl.ds(s * SLICE, SLICE)], red.at[t])

        def rb(g, _):
            v = red[0, pl.ds(g * LANES, LANES)]
            for t in range(1, NS):
                v = v + red[t, pl.ds(g * LANES, LANES)]
            rbuf[pl.ds(g * LANES, LANES)] = group_fn(g, v)
            return 0
        lax.fori_loop(0, SLICE // LANES, rb, 0)
        plsc.subcore_barrier()

    def broadcast_rbuf():
        pltpu.sync_copy(rbuf, red_sh.at[pl.ds(s * SLICE, SLICE)])
        plsc.subcore_barrier()
        pltpu.sync_copy(red_sh, full)
        plsc.subcore_barrier()

    def init_w(j, _):
        r = real_of(j)
        for g in range(CHUNK // LANES):
            sl = pl.ds(g * LANES, LANES)
            w_t[j, sl] = jnp.full((LANES,), 1.0) * r
        return 0
    lax.fori_loop(0, CPT, init_w, 0)

    for it in range(5):
        zero_acc()

        def sc_body(j, _):
            for g in range(CHUNK // LANES):
                sl = pl.ds(g * LANES, LANES)
                w16 = w_t[j, sl]
                plsc.addupdate_scatter(acc, [src_t[j, sl]], w16)
                plsc.addupdate_scatter(acc, [dst_t[j, sl]], w16)
            return 0
        lax.fori_loop(0, CPT, sc_body, 0)
        stage_reduce(lambda g, v: v)
        broadcast_rbuf()

        def k_body(j, vs):
            for g in range(CHUNK // LANES):
                sl = pl.ds(g * LANES, LANES)
                a = plsc.load_gather(full, [src_t[j, sl]])
                b = plsc.load_gather(full, [dst_t[j, sl]])
                e2 = jnp.exp(0.2 * (4.0 - a - b))
                th = (e2 - 1.0) / (e2 + 1.0)
                w16 = w_t[j, sl] * (1.0 - 0.05 * th)
                w_t[j, sl] = w16
                vs = vs + w16
            return vs
        vsum = lax.fori_loop(0, CPT, k_body, jnp.zeros((LANES,), jnp.float32))

        ssum[0, :] = vsum
        pltpu.sync_copy(ssum.at[0], sums_sh.at[s])
        plsc.subcore_barrier()
        pltpu.sync_copy(sums_sh, ssum)
        tot16 = ssum[0, :]
        for t in range(1, NS):
            tot16 = tot16 + ssum[t, :]
        total = jnp.sum(tot16)
        scale = (jnp.full((LANES,), float(E), jnp.float32)
                 / jnp.maximum(jnp.broadcast_to(total, (LANES,)), 1e-12))
        plsc.subcore_barrier()

        def renorm(j, _):
            r = real_of(j)
            for g in range(CHUNK // LANES):
                sl = pl.ds(g * LANES, LANES)
                w16 = jnp.maximum(w_t[j, sl] * scale, 1e-6) * r
                w_t[j, sl] = w16
                if it == 0:
                    w1_t[j, sl] = w16
            return 0
        lax.fori_loop(0, CPT, renorm, 0)

    for gi in range(3):
        def ew16(j, sl):
            if gi == 0:
                return jnp.full((LANES,), 1.0) * real_of(j)
            return (w1_t if gi == 1 else w_t)[j, sl]

        zero_acc()

        def d_body(j, _):
            for g in range(CHUNK // LANES):
                sl = pl.ds(g * LANES, LANES)
                plsc.addupdate_scatter(acc, [dst_t[j, sl]], ew16(j, sl))
            return 0
        lax.fori_loop(0, CPT, d_body, 0)
        stage_reduce(lambda g, v: _rsqrt16(v + 1.0))
        pltpu.sync_copy(rbuf, dinv3.at[gi].at[pl.ds(s * SLICE, SLICE)])
        broadcast_rbuf()

        zero_acc()

        def n_body(jj, _):
            for k in range(LANES):
                j = jj * LANES + k
                for g in range(CHUNK // LANES):
                    sl = pl.ds(g * LANES, LANES)
                    d16 = dst_t[j, sl]
                    a = plsc.load_gather(full, [src_t[j, sl]])
                    b = plsc.load_gather(full, [d16])
                    nm = a * ew16(j, sl) * b
                    nbuf[k, sl] = nm
                    plsc.addupdate_scatter(acc, [d16], nm)
            pltpu.sync_copy(
                nbuf, norm3.at[gi].at[pl.ds(base_row + jj * LANES, LANES)])
            return 0
        lax.fori_loop(0, CPT // LANES, n_body, 0)

        def r_group(g, v):
            d16 = full[pl.ds(s * SLICE + g * LANES, LANES)]
            return v + d16 * d16
        stage_reduce(r_group)
        pltpu.sync_copy(rbuf, r3.at[gi].at[pl.ds(s * SLICE, SLICE)])
        plsc.subcore_barrier()


@jax.jit
def _scalar_sc(srcb, dstb):
    mesh = plsc.VectorSubcoreMesh(
        core_axis_name="c", subcore_axis_name="s", num_cores=1)
    fn = pl.kernel(
        _scalar_body,
        out_type=(
            jax.ShapeDtypeStruct((3, NPAD), jnp.float32),
            jax.ShapeDtypeStruct((3, NPAD), jnp.float32),
            jax.ShapeDtypeStruct((3, NROWS, CHUNK), jnp.float32),
        ),
        mesh=mesh,
        scratch_types=[
            pltpu.VMEM((CPT, CHUNK), jnp.int32),
            pltpu.VMEM((CPT, CHUNK), jnp.int32),
            pltpu.VMEM((CPT, CHUNK), jnp.float32),
            pltpu.VMEM((CPT, CHUNK), jnp.float32),
            pltpu.VMEM((NPAD,), jnp.float32),
            pltpu.VMEM((NPAD,), jnp.float32),
            pltpu.VMEM((NS, SLICE), jnp.float32),
            pltpu.VMEM((SLICE,), jnp.float32),
            pltpu.VMEM((LANES, CHUNK), jnp.float32),
            pltpu.VMEM((NS, LANES), jnp.float32),
            pltpu.VMEM_SHARED((NS, NPAD), jnp.float32),
            pltpu.VMEM_SHARED((NPAD,), jnp.float32),
            pltpu.VMEM_SHARED((NS, LANES), jnp.float32),
        ],
        compiler_params=pltpu.CompilerParams(
            use_tc_tiling_on_sc=False, needs_layout_passes=False),
    )
    return fn(srcb, dstb)


def _propagate(x, srcb, dstb, normb, dinv):
    ch = x.shape[1]
    if ch > 64:
        edge = jnp.concatenate(
            [_propagate_sc(x[:, i:i + 64], srcb, dstb, normb, ch=64)
             for i in range(0, ch, 64)], axis=1)
    else:
        edge = _propagate_sc(x, srcb, dstb, normb, ch=ch)
    return edge + (dinv * dinv)[:, None] * x


def kernel(x, edge_index, W1, b1, Wmu, bmu, Wlv, blv, Wh, bh):
    src = edge_index[0]
    dst = edge_index[1]
    srcb, dstb = _pad_edges(src, dst)

    dinv3, r3, norm3 = _scalar_sc(srcb, dstb)

    Wc = jnp.concatenate([Wmu, Wlv], axis=1)

    dinv, r, normb = dinv3[0, :N], r3[0, :N], norm3[0]
    aggx = _propagate(x, srcb, dstb, normb, dinv)
    h = _matmul(aggx, W1, b1, r, act="relu")
    hc = _matmul(h, Wc, jnp.zeros((Wc.shape[1],), jnp.float32), r)
    agg_hc = _propagate(hc, srcb, dstb, normb, dinv)
    mu = agg_hc[:, : Wmu.shape[1]] + r[:, None] * bmu
    logvar = agg_hc[:, Wmu.shape[1]:] + r[:, None] * blv
    z = mu

    def encode_mu(gi):
        dinv_g, r_g, normb_g = dinv3[gi, :N], r3[gi, :N], norm3[gi]
        aggx_g = _propagate(x, srcb, dstb, normb_g, dinv_g)
        h_g = _matmul(aggx_g, W1, b1, r_g, act="relu")
        hm_g = _matmul(h_g, Wmu, jnp.zeros((Wmu.shape[1],), jnp.float32), r_g)
        return _propagate(hm_g, srcb, dstb, normb_g, dinv_g) + r_g[:, None] * bmu

    z1 = encode_mu(1)
    z5 = encode_mu(2)

    adj_pred, task_pred = _decode(z, Wh, bh)
    return (z, mu, logvar, z1, z5, adj_pred, task_pred)
